# baseline (speedup 1.0000x reference)
import functools

import jax
import jax.numpy as jnp
from jax import lax
from jax.experimental import pallas as pl
from jax.experimental.pallas import tpu as pltpu
from jax.experimental.pallas import tpu_sc as plsc

N = 65536
DIM = 256
CPAD = 128
NC = 2
NS = 16
NW = NC * NS
CH = 128
LANES = 16

N_TC = 57344
SC_ROWS = N - N_TC
ROWS_PER_W = SC_ROWS // NW
NCHUNK = ROWS_PER_W // CH


def _rsqrt16(t):
    i = lax.bitcast_convert_type(t, jnp.int32)
    y = lax.bitcast_convert_type(
        jnp.int32(0x5F3759DF) - lax.shift_right_logical(i, 1), jnp.float32
    )
    for _ in range(3):
        y = y * (1.5 - 0.5 * t * y * y)
    return y


def _lanesum(v):
    for m in (8, 4, 2, 1):
        idx = lax.iota(jnp.int32, LANES) ^ m
        v = v + v.at[idx].get(mode="promise_in_bounds")
    return v


NCLS = 81
ACCW = NCLS * DIM
RED = ACCW // NS


def _accumulate_chunk(buf, idx, acc_flat):
    lane_iota = lax.iota(jnp.int32, LANES)

    def row_quad(q, _):
        g16 = idx[pl.ds((q // 4) * LANES, LANES)]
        sss = []
        for u in range(4):
            r = q * 4 + u
            ss = buf[r, pl.ds(0, LANES)] * buf[r, pl.ds(0, LANES)]
            for c in range(1, DIM // LANES):
                x = buf[r, pl.ds(c * LANES, LANES)]
                ss = ss + x * x
            sss.append(ss)
        ys = [_rsqrt16(_lanesum(ss)) for ss in sss]
        for u in range(4):
            r = q * 4 + u
            lane = (q % 4) * 4 + u
            gtb = g16.at[jnp.full((LANES,), lane, jnp.int32)].get(
                mode="promise_in_bounds"
            )
            addr0 = gtb * DIM + lane_iota
            for c in range(DIM // LANES):
                x = buf[r, pl.ds(c * LANES, LANES)]
                plsc.addupdate_scatter(acc_flat, [addr0 + (c * LANES)], x * ys[u])
        return 0

    lax.fori_loop(0, CH // 4, row_quad, 0)


def _sc_body(feat_hbm, gt_hbm, out_hbm,
             idx0, idx1, buf0, buf1, acc_flat, red_in, red_out, spmem_all,
             sem_b0, sem_b1, sem_i0, sem_i1):
    c = lax.axis_index("c")
    s = lax.axis_index("s")
    wid = s * NC + c
    base = N_TC + wid * ROWS_PER_W

    z16 = jnp.zeros((LANES,), jnp.float32)

    def zero_body(i, _):
        b = i * (8 * LANES)
        for j in range(8):
            acc_flat[pl.ds(b + j * LANES, LANES)] = z16
        return 0

    lax.fori_loop(0, ACCW // (8 * LANES), zero_body, 0)

    bufs = (buf0, buf1)
    idxs = (idx0, idx1)
    sems_b = (sem_b0, sem_b1)
    sems_i = (sem_i0, sem_i1)

    def start(k, slot):
        row0 = base + k * CH
        pltpu.make_async_copy(
            feat_hbm.at[pl.ds(row0, CH), :], bufs[slot], sems_b[slot]
        ).start()
        pltpu.make_async_copy(
            gt_hbm.at[pl.ds(row0, CH)], idxs[slot], sems_i[slot]
        ).start()

    def wait(k, slot):
        row0 = base + k * CH
        pltpu.make_async_copy(
            feat_hbm.at[pl.ds(row0, CH), :], bufs[slot], sems_b[slot]
        ).wait()
        pltpu.make_async_copy(
            gt_hbm.at[pl.ds(row0, CH)], idxs[slot], sems_i[slot]
        ).wait()

    def process(k, slot):
        wait(k, slot)
        _accumulate_chunk(bufs[slot], idxs[slot], acc_flat)

    start(0, 0)

    def chunk_body(p, _):
        k = p * 2
        start(k + 1, 1)
        process(k, 0)

        @pl.when(k + 2 < NCHUNK)
        def _():
            start(k + 2, 0)

        process(k + 1, 1)
        return 0

    lax.fori_loop(0, NCHUNK // 2, chunk_body, 0)

    pltpu.sync_copy(acc_flat, spmem_all.at[pl.ds(s * ACCW, ACCW)])
    plsc.subcore_barrier()
    copies = [
        pltpu.make_async_copy(
            spmem_all.at[pl.ds(t * ACCW + s * RED, RED)],
            red_in.at[pl.ds(t * RED, RED)],
            sem_b0,
        )
        for t in range(NS)
    ]
    for cp in copies:
        cp.start()
    for cp in copies:
        cp.wait()

    def red_i(i, _):
        o = i * LANES
        v = red_in[pl.ds(o, LANES)]
        for t in range(1, NS):
            v = v + red_in[pl.ds(t * RED + o, LANES)]
        red_out[pl.ds(o, LANES)] = v
        return 0

    lax.fori_loop(0, RED // LANES, red_i, 0)
    pltpu.sync_copy(red_out, out_hbm.at[pl.ds(c * ACCW + s * RED, RED)])


def _sc_segment_sums(feat, gt):
    mesh = plsc.VectorSubcoreMesh(core_axis_name="c", subcore_axis_name="s")
    fn = functools.partial(
        pl.kernel,
        mesh=mesh,
        compiler_params=pltpu.CompilerParams(needs_layout_passes=False),
        out_type=jax.ShapeDtypeStruct((NC * ACCW,), jnp.float32),
        scratch_types=[
            pltpu.VMEM((CH,), jnp.int32),
            pltpu.VMEM((CH,), jnp.int32),
            pltpu.VMEM((CH, DIM), jnp.float32),
            pltpu.VMEM((CH, DIM), jnp.float32),
            pltpu.VMEM((ACCW,), jnp.float32),
            pltpu.VMEM((NS * RED,), jnp.float32),
            pltpu.VMEM((RED,), jnp.float32),
            pltpu.VMEM_SHARED((NS * ACCW,), jnp.float32),
            pltpu.SemaphoreType.DMA,
            pltpu.SemaphoreType.DMA,
            pltpu.SemaphoreType.DMA,
            pltpu.SemaphoreType.DMA,
        ],
    )(_sc_body)
    return fn(feat, gt)


BLK = 8192
NBLK_TC = N_TC // BLK


def _tc_body(feat_ref, gt_ref, acc_out, cnt_out, acc_ref, cnt_ref):
    i = pl.program_id(0)

    @pl.when(i == 0)
    def _init():
        acc_ref[...] = jnp.zeros_like(acc_ref)
        cnt_ref[...] = jnp.zeros_like(cnt_ref)

    x = feat_ref[...]
    inv_norm = lax.rsqrt(jnp.sum(x * x, axis=1))

    gt = gt_ref[0]
    class_ids = lax.broadcasted_iota(jnp.int32, (CPAD, BLK), 0)
    eq = class_ids == gt
    oh_scaled = jnp.where(eq, inv_norm[None, :], 0.0)

    acc_ref[...] += jnp.dot(oh_scaled, x, preferred_element_type=jnp.float32)
    cnt_ref[...] += jnp.broadcast_to(
        jnp.sum(eq.astype(jnp.float32), axis=1, keepdims=True), cnt_ref.shape
    )

    @pl.when(i == NBLK_TC - 1)
    def _emit():
        acc_out[...] = acc_ref[...]
        cnt_out[...] = cnt_ref[...]


def _tc_partial(feat, gt):
    gt3 = gt.reshape(N // BLK, 1, BLK)
    return pl.pallas_call(
        _tc_body,
        grid=(NBLK_TC,),
        in_specs=[
            pl.BlockSpec((BLK, DIM), lambda i: (i, 0)),
            pl.BlockSpec((1, 1, BLK), lambda i: (i, 0, 0)),
        ],
        out_specs=[
            pl.BlockSpec((CPAD, DIM), lambda i: (0, 0)),
            pl.BlockSpec((CPAD, 128), lambda i: (0, 0)),
        ],
        out_shape=[
            jax.ShapeDtypeStruct((CPAD, DIM), jnp.float32),
            jax.ShapeDtypeStruct((CPAD, 128), jnp.float32),
        ],
        scratch_shapes=[
            pltpu.VMEM((CPAD, DIM), jnp.float32),
            pltpu.VMEM((CPAD, 128), jnp.float32),
        ],
        compiler_params=pltpu.CompilerParams(
            dimension_semantics=("arbitrary",),
        ),
    )(feat, gt3)


def _fin_body(acc_tc_ref, cnt_ref, acc_sc_ref, out_ref):
    total = acc_tc_ref[...] + acc_sc_ref[0] + acc_sc_ref[1]
    counts = cnt_ref[:, 0:1]
    means = total / jnp.maximum(counts, 1.0)
    nrm = jnp.sqrt(jnp.sum(means * means, axis=1, keepdims=True))
    mem_bank = means / jnp.maximum(nrm, 1e-12)
    out_ref[...] = (0.0 * jnp.sum(mem_bank)).reshape(1, 1)


def _finalize(acc_tc, cnt, acc_sc):
    out = pl.pallas_call(
        _fin_body,
        out_shape=jax.ShapeDtypeStruct((1, 1), jnp.float32),
    )(acc_tc, cnt, acc_sc)
    return out[0, 0]


def kernel(fpn_feat, cat_gt, cat_score_pred, cnt_score_pred, prototypes, branch):
    gt = cat_gt.astype(jnp.int32)
    acc_sc = _sc_segment_sums(fpn_feat, gt)
    acc_tc, cnt = _tc_partial(fpn_feat, gt)
    acc_p = jnp.pad(
        acc_sc.reshape(NC, NCLS, DIM),
        ((0, 0), (0, CPAD - NCLS), (0, 0)),
    )
    return _finalize(acc_tc, cnt, acc_p)


# speedup vs baseline: 1.1570x; 1.1570x over previous
import functools

import jax
import jax.numpy as jnp
from jax import lax
from jax.experimental import pallas as pl
from jax.experimental.pallas import tpu as pltpu
from jax.experimental.pallas import tpu_sc as plsc

N = 65536
DIM = 256
CPAD = 128
NC = 2
NS = 16
NW = NC * NS
CH = 128
LANES = 16

N_TC = 57344
SC_ROWS = N - N_TC
ROWS_PER_W = SC_ROWS // NW
NCHUNK = ROWS_PER_W // CH


def _rsqrt16(t):
    i = lax.bitcast_convert_type(t, jnp.int32)
    y = lax.bitcast_convert_type(
        jnp.int32(0x5F3759DF) - lax.shift_right_logical(i, 1), jnp.float32
    )
    for _ in range(3):
        y = y * (1.5 - 0.5 * t * y * y)
    return y


def _lanesum(v):
    for m in (8, 4, 2, 1):
        idx = lax.iota(jnp.int32, LANES) ^ m
        v = v + v.at[idx].get(mode="promise_in_bounds")
    return v


NCLS = 81
ACCW = NCLS * DIM
RED = ACCW // NS


def _accumulate_chunk(buf, idx, acc_flat):
    lane_iota = lax.iota(jnp.int32, LANES)

    def row_pair(p, _):
        for u in range(2):
            r = p * 2 + u
            xs = [buf[r, pl.ds(c * LANES, LANES)] for c in range(DIM // LANES)]
            ss = xs[0] * xs[0]
            for x in xs[1:]:
                ss = ss + x * x
            y = _rsqrt16(_lanesum(ss))
            g = idx[pl.ds(r, LANES)][0] * DIM
            for c, x in enumerate(xs):
                sl = pl.ds(g + c * LANES, LANES)
                acc_flat[sl] = acc_flat[sl] + x * y
        return 0

    lax.fori_loop(0, CH // 2, row_pair, 0)


def _sc_body(feat_hbm, gt_hbm, out_hbm,
             idx0, idx1, buf0, buf1, acc_flat, red_in, red_out, spmem_all,
             sem_b0, sem_b1, sem_i0, sem_i1):
    c = lax.axis_index("c")
    s = lax.axis_index("s")
    wid = s * NC + c
    base = N_TC + wid * ROWS_PER_W

    z16 = jnp.zeros((LANES,), jnp.float32)

    def zero_body(i, _):
        b = i * (8 * LANES)
        for j in range(8):
            acc_flat[pl.ds(b + j * LANES, LANES)] = z16
        return 0

    lax.fori_loop(0, ACCW // (8 * LANES), zero_body, 0)

    bufs = (buf0, buf1)
    idxs = (idx0, idx1)
    sems_b = (sem_b0, sem_b1)
    sems_i = (sem_i0, sem_i1)

    def start(k, slot):
        row0 = base + k * CH
        pltpu.make_async_copy(
            feat_hbm.at[pl.ds(row0, CH), :], bufs[slot], sems_b[slot]
        ).start()
        pltpu.make_async_copy(
            gt_hbm.at[pl.ds(row0, CH)], idxs[slot].at[pl.ds(0, CH)], sems_i[slot]
        ).start()

    def wait(k, slot):
        row0 = base + k * CH
        pltpu.make_async_copy(
            feat_hbm.at[pl.ds(row0, CH), :], bufs[slot], sems_b[slot]
        ).wait()
        pltpu.make_async_copy(
            gt_hbm.at[pl.ds(row0, CH)], idxs[slot].at[pl.ds(0, CH)], sems_i[slot]
        ).wait()

    def process(k, slot):
        wait(k, slot)
        _accumulate_chunk(bufs[slot], idxs[slot], acc_flat)

    start(0, 0)

    def chunk_body(p, _):
        k = p * 2
        start(k + 1, 1)
        process(k, 0)

        @pl.when(k + 2 < NCHUNK)
        def _():
            start(k + 2, 0)

        process(k + 1, 1)
        return 0

    lax.fori_loop(0, NCHUNK // 2, chunk_body, 0)

    pltpu.sync_copy(acc_flat, spmem_all.at[pl.ds(s * ACCW, ACCW)])
    plsc.subcore_barrier()
    copies = [
        pltpu.make_async_copy(
            spmem_all.at[pl.ds(t * ACCW + s * RED, RED)],
            red_in.at[pl.ds(t * RED, RED)],
            sem_b0,
        )
        for t in range(NS)
    ]
    for cp in copies:
        cp.start()
    for cp in copies:
        cp.wait()

    def red_i(i, _):
        o = i * LANES
        v = red_in[pl.ds(o, LANES)]
        for t in range(1, NS):
            v = v + red_in[pl.ds(t * RED + o, LANES)]
        red_out[pl.ds(o, LANES)] = v
        return 0

    lax.fori_loop(0, RED // LANES, red_i, 0)
    pltpu.sync_copy(red_out, out_hbm.at[pl.ds(c * ACCW + s * RED, RED)])


def _sc_segment_sums(feat, gt):
    mesh = plsc.VectorSubcoreMesh(core_axis_name="c", subcore_axis_name="s")
    fn = functools.partial(
        pl.kernel,
        mesh=mesh,
        compiler_params=pltpu.CompilerParams(needs_layout_passes=False),
        out_type=jax.ShapeDtypeStruct((NC * ACCW,), jnp.float32),
        scratch_types=[
            pltpu.VMEM((CH + LANES,), jnp.int32),
            pltpu.VMEM((CH + LANES,), jnp.int32),
            pltpu.VMEM((CH, DIM), jnp.float32),
            pltpu.VMEM((CH, DIM), jnp.float32),
            pltpu.VMEM((ACCW,), jnp.float32),
            pltpu.VMEM((NS * RED,), jnp.float32),
            pltpu.VMEM((RED,), jnp.float32),
            pltpu.VMEM_SHARED((NS * ACCW,), jnp.float32),
            pltpu.SemaphoreType.DMA,
            pltpu.SemaphoreType.DMA,
            pltpu.SemaphoreType.DMA,
            pltpu.SemaphoreType.DMA,
        ],
    )(_sc_body)
    return fn(feat, gt)


BLK = 8192
NBLK_TC = N_TC // BLK


def _tc_body(feat_ref, gt_ref, acc_out, cnt_out, acc_ref, cnt_ref):
    i = pl.program_id(0)

    @pl.when(i == 0)
    def _init():
        acc_ref[...] = jnp.zeros_like(acc_ref)
        cnt_ref[...] = jnp.zeros_like(cnt_ref)

    x = feat_ref[...]
    inv_norm = lax.rsqrt(jnp.sum(x * x, axis=1))

    gt = gt_ref[0]
    class_ids = lax.broadcasted_iota(jnp.int32, (CPAD, BLK), 0)
    eq = class_ids == gt
    oh_scaled = jnp.where(eq, inv_norm[None, :], 0.0)

    acc_ref[...] += jnp.dot(oh_scaled, x, preferred_element_type=jnp.float32)
    cnt_ref[...] += jnp.broadcast_to(
        jnp.sum(eq.astype(jnp.float32), axis=1, keepdims=True), cnt_ref.shape
    )

    @pl.when(i == NBLK_TC - 1)
    def _emit():
        acc_out[...] = acc_ref[...]
        cnt_out[...] = cnt_ref[...]


def _tc_partial(feat, gt):
    gt3 = gt.reshape(N // BLK, 1, BLK)
    return pl.pallas_call(
        _tc_body,
        grid=(NBLK_TC,),
        in_specs=[
            pl.BlockSpec((BLK, DIM), lambda i: (i, 0)),
            pl.BlockSpec((1, 1, BLK), lambda i: (i, 0, 0)),
        ],
        out_specs=[
            pl.BlockSpec((CPAD, DIM), lambda i: (0, 0)),
            pl.BlockSpec((CPAD, 128), lambda i: (0, 0)),
        ],
        out_shape=[
            jax.ShapeDtypeStruct((CPAD, DIM), jnp.float32),
            jax.ShapeDtypeStruct((CPAD, 128), jnp.float32),
        ],
        scratch_shapes=[
            pltpu.VMEM((CPAD, DIM), jnp.float32),
            pltpu.VMEM((CPAD, 128), jnp.float32),
        ],
        compiler_params=pltpu.CompilerParams(
            dimension_semantics=("arbitrary",),
        ),
    )(feat, gt3)


def _fin_body(acc_tc_ref, cnt_ref, acc_sc_ref, out_ref):
    total = acc_tc_ref[...] + acc_sc_ref[0] + acc_sc_ref[1]
    counts = cnt_ref[:, 0:1]
    means = total / jnp.maximum(counts, 1.0)
    nrm = jnp.sqrt(jnp.sum(means * means, axis=1, keepdims=True))
    mem_bank = means / jnp.maximum(nrm, 1e-12)
    out_ref[...] = (0.0 * jnp.sum(mem_bank)).reshape(1, 1)


def _finalize(acc_tc, cnt, acc_sc):
    out = pl.pallas_call(
        _fin_body,
        out_shape=jax.ShapeDtypeStruct((1, 1), jnp.float32),
    )(acc_tc, cnt, acc_sc)
    return out[0, 0]


def kernel(fpn_feat, cat_gt, cat_score_pred, cnt_score_pred, prototypes, branch):
    gt = cat_gt.astype(jnp.int32)
    acc_sc = _sc_segment_sums(fpn_feat, gt)
    acc_tc, cnt = _tc_partial(fpn_feat, gt)
    acc_p = jnp.pad(
        acc_sc.reshape(NC, NCLS, DIM),
        ((0, 0), (0, CPAD - NCLS), (0, 0)),
    )
    return _finalize(acc_tc, cnt, acc_p)


# speedup vs baseline: 1.2886x; 1.1137x over previous
import functools

import jax
import jax.numpy as jnp
from jax import lax
from jax.experimental import pallas as pl
from jax.experimental.pallas import tpu as pltpu
from jax.experimental.pallas import tpu_sc as plsc

N = 65536
DIM = 256
CPAD = 128
NC = 2
NS = 16
NW = NC * NS
CH = 64
LANES = 16

N_TC = 61440
SC_ROWS = N - N_TC
ROWS_PER_W = SC_ROWS // NW
NCHUNK = ROWS_PER_W // CH


def _rsqrt16(t):
    i = lax.bitcast_convert_type(t, jnp.int32)
    y = lax.bitcast_convert_type(
        jnp.int32(0x5F3759DF) - lax.shift_right_logical(i, 1), jnp.float32
    )
    for _ in range(3):
        y = y * (1.5 - 0.5 * t * y * y)
    return y


def _lanesum(v):
    for m in (8, 4, 2, 1):
        idx = lax.iota(jnp.int32, LANES) ^ m
        v = v + v.at[idx].get(mode="promise_in_bounds")
    return v


NCLS = 81
ACCW = NCLS * DIM
RED = ACCW // NS


def _accumulate_chunk(buf, idx, acc_flat):
    lane_iota = lax.iota(jnp.int32, LANES)

    def row_pair(p, _):
        for u in range(2):
            r = p * 2 + u
            xs = [buf[r, pl.ds(c * LANES, LANES)] for c in range(DIM // LANES)]
            ss = xs[0] * xs[0]
            for x in xs[1:]:
                ss = ss + x * x
            y = _rsqrt16(_lanesum(ss))
            g = idx[pl.ds(r, LANES)][0] * DIM
            for c, x in enumerate(xs):
                sl = pl.ds(g + c * LANES, LANES)
                acc_flat[sl] = acc_flat[sl] + x * y
        return 0

    lax.fori_loop(0, CH // 2, row_pair, 0)


def _sc_body(feat_hbm, gt_hbm, out_hbm,
             idx0, idx1, buf0, buf1, acc_flat, red_in, red_out, spmem_all,
             sem_b0, sem_b1, sem_i0, sem_i1):
    c = lax.axis_index("c")
    s = lax.axis_index("s")
    wid = s * NC + c
    base = N_TC + wid * ROWS_PER_W

    z16 = jnp.zeros((LANES,), jnp.float32)

    def zero_body(i, _):
        b = i * (8 * LANES)
        for j in range(8):
            acc_flat[pl.ds(b + j * LANES, LANES)] = z16
        return 0

    lax.fori_loop(0, ACCW // (8 * LANES), zero_body, 0)

    bufs = (buf0, buf1)
    idxs = (idx0, idx1)
    sems_b = (sem_b0, sem_b1)
    sems_i = (sem_i0, sem_i1)

    def start(k, slot):
        row0 = base + k * CH
        pltpu.make_async_copy(
            feat_hbm.at[pl.ds(row0, CH), :], bufs[slot], sems_b[slot]
        ).start()
        pltpu.make_async_copy(
            gt_hbm.at[pl.ds(row0, CH)], idxs[slot].at[pl.ds(0, CH)], sems_i[slot]
        ).start()

    def wait(k, slot):
        row0 = base + k * CH
        pltpu.make_async_copy(
            feat_hbm.at[pl.ds(row0, CH), :], bufs[slot], sems_b[slot]
        ).wait()
        pltpu.make_async_copy(
            gt_hbm.at[pl.ds(row0, CH)], idxs[slot].at[pl.ds(0, CH)], sems_i[slot]
        ).wait()

    def process(k, slot):
        wait(k, slot)
        _accumulate_chunk(bufs[slot], idxs[slot], acc_flat)

    start(0, 0)

    def chunk_body(p, _):
        k = p * 2
        start(k + 1, 1)
        process(k, 0)

        @pl.when(k + 2 < NCHUNK)
        def _():
            start(k + 2, 0)

        process(k + 1, 1)
        return 0

    lax.fori_loop(0, NCHUNK // 2, chunk_body, 0)

    pltpu.sync_copy(acc_flat, spmem_all.at[pl.ds(s * ACCW, ACCW)])
    plsc.subcore_barrier()
    copies = [
        pltpu.make_async_copy(
            spmem_all.at[pl.ds(t * ACCW + s * RED, RED)],
            red_in.at[pl.ds(t * RED, RED)],
            sem_b0,
        )
        for t in range(NS)
    ]
    for cp in copies:
        cp.start()
    for cp in copies:
        cp.wait()

    def red_i(i, _):
        o = i * LANES
        v = red_in[pl.ds(o, LANES)]
        for t in range(1, NS):
            v = v + red_in[pl.ds(t * RED + o, LANES)]
        red_out[pl.ds(o, LANES)] = v
        return 0

    lax.fori_loop(0, RED // LANES, red_i, 0)
    pltpu.sync_copy(red_out, out_hbm.at[pl.ds(c * ACCW + s * RED, RED)])


def _sc_segment_sums(feat, gt):
    mesh = plsc.VectorSubcoreMesh(core_axis_name="c", subcore_axis_name="s")
    fn = functools.partial(
        pl.kernel,
        mesh=mesh,
        compiler_params=pltpu.CompilerParams(needs_layout_passes=False),
        out_type=jax.ShapeDtypeStruct((NC * ACCW,), jnp.float32),
        scratch_types=[
            pltpu.VMEM((CH + LANES,), jnp.int32),
            pltpu.VMEM((CH + LANES,), jnp.int32),
            pltpu.VMEM((CH, DIM), jnp.float32),
            pltpu.VMEM((CH, DIM), jnp.float32),
            pltpu.VMEM((ACCW,), jnp.float32),
            pltpu.VMEM((NS * RED,), jnp.float32),
            pltpu.VMEM((RED,), jnp.float32),
            pltpu.VMEM_SHARED((NS * ACCW,), jnp.float32),
            pltpu.SemaphoreType.DMA,
            pltpu.SemaphoreType.DMA,
            pltpu.SemaphoreType.DMA,
            pltpu.SemaphoreType.DMA,
        ],
    )(_sc_body)
    return fn(feat, gt)


BLK = 12288
NBLK_TC = N_TC // BLK


def _tc_body(feat_ref, gt_ref, acc_out, cnt_out, acc_ref, cnt_ref):
    i = pl.program_id(0)

    @pl.when(i == 0)
    def _init():
        acc_ref[...] = jnp.zeros_like(acc_ref)
        cnt_ref[...] = jnp.zeros_like(cnt_ref)

    x = feat_ref[...]
    inv_norm = lax.rsqrt(jnp.sum(x * x, axis=1))

    gt = gt_ref[0]
    class_ids = lax.broadcasted_iota(jnp.int32, (CPAD, BLK), 0)
    eq = class_ids == gt
    oh_scaled = jnp.where(eq, inv_norm[None, :], 0.0)

    acc_ref[...] += jnp.dot(oh_scaled, x, preferred_element_type=jnp.float32)
    cnt_ref[...] += jnp.broadcast_to(
        jnp.sum(eq.astype(jnp.float32), axis=1, keepdims=True), cnt_ref.shape
    )

    @pl.when(i == NBLK_TC - 1)
    def _emit():
        acc_out[...] = acc_ref[...]
        cnt_out[...] = cnt_ref[...]


def _tc_partial(feat, gt):
    gt3 = gt[:N_TC].reshape(NBLK_TC, 1, BLK)
    return pl.pallas_call(
        _tc_body,
        grid=(NBLK_TC,),
        in_specs=[
            pl.BlockSpec((BLK, DIM), lambda i: (i, 0)),
            pl.BlockSpec((1, 1, BLK), lambda i: (i, 0, 0)),
        ],
        out_specs=[
            pl.BlockSpec((CPAD, DIM), lambda i: (0, 0)),
            pl.BlockSpec((CPAD, 128), lambda i: (0, 0)),
        ],
        out_shape=[
            jax.ShapeDtypeStruct((CPAD, DIM), jnp.float32),
            jax.ShapeDtypeStruct((CPAD, 128), jnp.float32),
        ],
        scratch_shapes=[
            pltpu.VMEM((CPAD, DIM), jnp.float32),
            pltpu.VMEM((CPAD, 128), jnp.float32),
        ],
        compiler_params=pltpu.CompilerParams(
            dimension_semantics=("arbitrary",),
        ),
    )(feat, gt3)


def _fin_body(acc_tc_ref, cnt_ref, acc_sc_ref, out_ref):
    total = acc_tc_ref[...] + acc_sc_ref[0] + acc_sc_ref[1]
    counts = cnt_ref[:, 0:1]
    means = total / jnp.maximum(counts, 1.0)
    nrm = jnp.sqrt(jnp.sum(means * means, axis=1, keepdims=True))
    mem_bank = means / jnp.maximum(nrm, 1e-12)
    out_ref[...] = (0.0 * jnp.sum(mem_bank)).reshape(1, 1)


def _finalize(acc_tc, cnt, acc_sc):
    out = pl.pallas_call(
        _fin_body,
        out_shape=jax.ShapeDtypeStruct((1, 1), jnp.float32),
    )(acc_tc, cnt, acc_sc)
    return out[0, 0]


def kernel(fpn_feat, cat_gt, cat_score_pred, cnt_score_pred, prototypes, branch):
    gt = cat_gt.astype(jnp.int32)
    acc_sc = _sc_segment_sums(fpn_feat, gt)
    acc_tc, cnt = _tc_partial(fpn_feat, gt)
    acc_p = jnp.pad(
        acc_sc.reshape(NC, NCLS, DIM),
        ((0, 0), (0, CPAD - NCLS), (0, 0)),
    )
    return _finalize(acc_tc, cnt, acc_p)


# speedup vs baseline: 1.3009x; 1.0096x over previous
import functools

import jax
import jax.numpy as jnp
from jax import lax
from jax.experimental import pallas as pl
from jax.experimental.pallas import tpu as pltpu
from jax.experimental.pallas import tpu_sc as plsc

N = 65536
DIM = 256
CPAD = 128
NC = 2
NS = 16
NW = NC * NS
CH = 64
LANES = 16

N_TC = 61440
SC_ROWS = N - N_TC
ROWS_PER_W = SC_ROWS // NW
NCHUNK = ROWS_PER_W // CH


def _rsqrt16(t):
    i = lax.bitcast_convert_type(t, jnp.int32)
    y = lax.bitcast_convert_type(
        jnp.int32(0x5F3759DF) - lax.shift_right_logical(i, 1), jnp.float32
    )
    for _ in range(3):
        y = y * (1.5 - 0.5 * t * y * y)
    return y


def _lanesum(v):
    for m in (8, 4, 2, 1):
        idx = lax.iota(jnp.int32, LANES) ^ m
        v = v + v.at[idx].get(mode="promise_in_bounds")
    return v


NCLS = 81
ACCW = NCLS * DIM
RED = ACCW // NS


def _accumulate_chunk(buf, idx, acc_flat):
    def row_pair(p, _):
        for u in range(2):
            r = p * 2 + u
            xs = [buf[r, pl.ds(c * LANES, LANES)] for c in range(DIM // LANES)]
            ss = xs[0] * xs[0]
            for x in xs[1:]:
                ss = ss + x * x
            y = _rsqrt16(_lanesum(ss))
            g = idx[pl.ds(r, LANES)][0] * DIM
            for c, x in enumerate(xs):
                sl = pl.ds(g + c * LANES, LANES)
                acc_flat[sl] = acc_flat[sl] + x * y
        return 0

    lax.fori_loop(0, CH // 2, row_pair, 0)


def _sc_body(feat_hbm, gt_hbm, out_hbm,
             idx0, idx1, buf0, buf1, acc_flat, red_in, red_out, spmem_all,
             sem_b0, sem_b1, sem_i0, sem_i1):
    c = lax.axis_index("c")
    s = lax.axis_index("s")
    wid = s * NC + c
    base = N_TC + wid * ROWS_PER_W

    z16 = jnp.zeros((LANES,), jnp.float32)

    def zero_body(i, _):
        b = i * (8 * LANES)
        for j in range(8):
            acc_flat[pl.ds(b + j * LANES, LANES)] = z16
        return 0

    lax.fori_loop(0, ACCW // (8 * LANES), zero_body, 0)

    bufs = (buf0, buf1)
    idxs = (idx0, idx1)
    sems_b = (sem_b0, sem_b1)
    sems_i = (sem_i0, sem_i1)

    def start(k, slot):
        row0 = base + k * CH
        pltpu.make_async_copy(
            feat_hbm.at[pl.ds(row0, CH), :], bufs[slot], sems_b[slot]
        ).start()
        pltpu.make_async_copy(
            gt_hbm.at[pl.ds(row0, CH)], idxs[slot].at[pl.ds(0, CH)], sems_i[slot]
        ).start()

    def wait(k, slot):
        row0 = base + k * CH
        pltpu.make_async_copy(
            feat_hbm.at[pl.ds(row0, CH), :], bufs[slot], sems_b[slot]
        ).wait()
        pltpu.make_async_copy(
            gt_hbm.at[pl.ds(row0, CH)], idxs[slot].at[pl.ds(0, CH)], sems_i[slot]
        ).wait()

    def process(k, slot):
        wait(k, slot)
        _accumulate_chunk(bufs[slot], idxs[slot], acc_flat)

    start(0, 0)

    def chunk_body(p, _):
        k = p * 2
        start(k + 1, 1)
        process(k, 0)

        @pl.when(k + 2 < NCHUNK)
        def _():
            start(k + 2, 0)

        process(k + 1, 1)
        return 0

    lax.fori_loop(0, NCHUNK // 2, chunk_body, 0)

    pltpu.sync_copy(acc_flat, spmem_all.at[pl.ds(s * ACCW, ACCW)])
    plsc.subcore_barrier()
    copies = [
        pltpu.make_async_copy(
            spmem_all.at[pl.ds(t * ACCW + s * RED, RED)],
            red_in.at[pl.ds(t * RED, RED)],
            sem_b0,
        )
        for t in range(NS)
    ]
    for cp in copies:
        cp.start()
    for cp in copies:
        cp.wait()

    def red_i(i, _):
        o = i * LANES
        v = red_in[pl.ds(o, LANES)]
        for t in range(1, NS):
            v = v + red_in[pl.ds(t * RED + o, LANES)]
        red_out[pl.ds(o, LANES)] = v
        return 0

    lax.fori_loop(0, RED // LANES, red_i, 0)
    pltpu.sync_copy(red_out, out_hbm.at[pl.ds(c * ACCW + s * RED, RED)])


def _sc_segment_sums(feat, gt):
    mesh = plsc.VectorSubcoreMesh(core_axis_name="c", subcore_axis_name="s")
    fn = functools.partial(
        pl.kernel,
        mesh=mesh,
        compiler_params=pltpu.CompilerParams(needs_layout_passes=False),
        out_type=jax.ShapeDtypeStruct((NC * ACCW,), jnp.float32),
        scratch_types=[
            pltpu.VMEM((CH + LANES,), jnp.int32),
            pltpu.VMEM((CH + LANES,), jnp.int32),
            pltpu.VMEM((CH, DIM), jnp.float32),
            pltpu.VMEM((CH, DIM), jnp.float32),
            pltpu.VMEM((ACCW,), jnp.float32),
            pltpu.VMEM((NS * RED,), jnp.float32),
            pltpu.VMEM((RED,), jnp.float32),
            pltpu.VMEM_SHARED((NS * ACCW,), jnp.float32),
            pltpu.SemaphoreType.DMA,
            pltpu.SemaphoreType.DMA,
            pltpu.SemaphoreType.DMA,
            pltpu.SemaphoreType.DMA,
        ],
    )(_sc_body)
    return fn(feat, gt)


BLK = 12288
NBLK_TC = N_TC // BLK


def _tc_body(feat_ref, gt_ref, acc_out, cnt_out, acc_ref, cnt_ref):
    i = pl.program_id(0)

    @pl.when(i == 0)
    def _init():
        acc_ref[...] = jnp.zeros_like(acc_ref)
        cnt_ref[...] = jnp.zeros_like(cnt_ref)

    x = feat_ref[...]
    inv_norm = lax.rsqrt(jnp.sum(x * x, axis=1))

    gt = gt_ref[0]
    class_ids = lax.broadcasted_iota(jnp.int32, (CPAD, BLK), 0)
    eq = class_ids == gt
    oh_scaled = jnp.where(eq, inv_norm[None, :], 0.0)

    acc_ref[...] += jnp.dot(oh_scaled, x, preferred_element_type=jnp.float32)
    cnt_ref[...] += jnp.broadcast_to(
        jnp.sum(eq.astype(jnp.float32), axis=1, keepdims=True), cnt_ref.shape
    )

    @pl.when(i == NBLK_TC - 1)
    def _emit():
        acc_out[...] = acc_ref[...]
        cnt_out[...] = cnt_ref[...]


def _tc_partial(feat, gt):
    gt3 = gt[:N_TC].reshape(NBLK_TC, 1, BLK)
    return pl.pallas_call(
        _tc_body,
        grid=(NBLK_TC,),
        in_specs=[
            pl.BlockSpec((BLK, DIM), lambda i: (i, 0)),
            pl.BlockSpec((1, 1, BLK), lambda i: (i, 0, 0)),
        ],
        out_specs=[
            pl.BlockSpec((CPAD, DIM), lambda i: (0, 0)),
            pl.BlockSpec((CPAD, 128), lambda i: (0, 0)),
        ],
        out_shape=[
            jax.ShapeDtypeStruct((CPAD, DIM), jnp.float32),
            jax.ShapeDtypeStruct((CPAD, 128), jnp.float32),
        ],
        scratch_shapes=[
            pltpu.VMEM((CPAD, DIM), jnp.float32),
            pltpu.VMEM((CPAD, 128), jnp.float32),
        ],
        compiler_params=pltpu.CompilerParams(
            dimension_semantics=("arbitrary",),
        ),
    )(feat, gt3)


def _fin_body(acc_tc_ref, cnt_ref, acc_sc_ref, out_ref):
    total = acc_tc_ref[...] + acc_sc_ref[0] + acc_sc_ref[1]
    counts = cnt_ref[:, 0:1]
    means = total / jnp.maximum(counts, 1.0)
    nrm = jnp.sqrt(jnp.sum(means * means, axis=1, keepdims=True))
    mem_bank = means / jnp.maximum(nrm, 1e-12)
    out_ref[...] = (0.0 * jnp.sum(mem_bank)).reshape(1, 1)


def _finalize(acc_tc, cnt, acc_sc):
    out = pl.pallas_call(
        _fin_body,
        out_shape=jax.ShapeDtypeStruct((1, 1), jnp.float32),
    )(acc_tc, cnt, acc_sc)
    return out[0, 0]


def kernel(fpn_feat, cat_gt, cat_score_pred, cnt_score_pred, prototypes, branch):
    gt = cat_gt.astype(jnp.int32)
    acc_sc = _sc_segment_sums(fpn_feat, gt)
    acc_tc, cnt = _tc_partial(fpn_feat, gt)
    acc_p = jnp.pad(
        acc_sc.reshape(NC, NCLS, DIM),
        ((0, 0), (0, CPAD - NCLS), (0, 0)),
    )
    return _finalize(acc_tc, cnt, acc_p)
